# Initial kernel scaffold; baseline (speedup 1.0000x reference)
import jax
import jax.numpy as jnp
from jax.experimental import pallas as pl


def kernel(token_ids, word_embeddings, pos_embeddings):
    raise NotImplementedError("write your pallas kernel here")


# speedup vs baseline: 1.3486x; 1.3486x over previous
import functools

import jax
import jax.numpy as jnp
from jax import lax
from jax.experimental import pallas as pl
from jax.experimental.pallas import tpu as pltpu
from jax.experimental.pallas import tpu_sc as plsc

_NUM_CORES = 2
_NUM_SUBCORES = 16
_NUM_WORKERS = _NUM_CORES * _NUM_SUBCORES


@functools.lru_cache(maxsize=None)
def _make_embed(batch, ctx, dim, n_chunks, chunk):
    rows_per_w = n_chunks * chunk
    mesh = plsc.VectorSubcoreMesh(
        core_axis_name="c",
        subcore_axis_name="s",
        num_cores=_NUM_CORES,
        num_subcores=_NUM_SUBCORES,
    )

    @functools.partial(
        pl.kernel,
        out_type=jax.ShapeDtypeStruct((batch * ctx, dim), jnp.float32),
        mesh=mesh,
        scratch_types=[
            pltpu.VMEM((n_chunks, chunk), jnp.int32),
            pltpu.VMEM((rows_per_w, dim), jnp.float32),
            pltpu.SemaphoreType.DMA,
        ],
    )
    def body(idx_hbm, table_hbm, pos_hbm, out_hbm, idx_v, acc_v, sem):
        wid = lax.axis_index("s") * _NUM_CORES + lax.axis_index("c")
        base = wid * rows_per_w
        pos_base = lax.rem(base, ctx)
        pltpu.sync_copy(idx_hbm.at[wid], idx_v)
        pltpu.sync_copy(pos_hbm.at[pl.ds(pos_base, rows_per_w)], acc_v)
        copies = [
            pltpu.async_copy(
                table_hbm.at[idx_v.at[j]],
                acc_v.at[pl.ds(j * chunk, chunk)],
                sem,
                add=True,
            )
            for j in range(n_chunks)
        ]
        for c in copies:
            c.wait()
        pltpu.sync_copy(acc_v, out_hbm.at[pl.ds(base, rows_per_w)])

    return body


def kernel(token_ids, word_embeddings, pos_embeddings):
    batch, ctx = token_ids.shape
    _, dim = word_embeddings.shape
    total = batch * ctx
    rows_per_w = total // _NUM_WORKERS
    chunk = min(128, rows_per_w)
    n_chunks = rows_per_w // chunk
    idx = token_ids.astype(jnp.int32).reshape(_NUM_WORKERS, n_chunks, chunk)
    fn = _make_embed(batch, ctx, dim, n_chunks, chunk)
    out = fn(idx, word_embeddings.astype(jnp.float32),
             pos_embeddings.astype(jnp.float32))
    return out.reshape(batch, ctx, dim)


# speedup vs baseline: 1.3646x; 1.0119x over previous
import functools

import jax
import jax.numpy as jnp
from jax import lax
from jax.experimental import pallas as pl
from jax.experimental.pallas import tpu as pltpu
from jax.experimental.pallas import tpu_sc as plsc

_NUM_CORES = 2
_NUM_SUBCORES = 16
_NUM_WORKERS = _NUM_CORES * _NUM_SUBCORES


@functools.lru_cache(maxsize=None)
def _make_embed(batch, ctx, dim, n_blocks, blk):
    rows_per_w = n_blocks * blk
    mesh = plsc.VectorSubcoreMesh(
        core_axis_name="c",
        subcore_axis_name="s",
        num_cores=_NUM_CORES,
        num_subcores=_NUM_SUBCORES,
    )

    @functools.partial(
        pl.kernel,
        out_type=jax.ShapeDtypeStruct((batch * ctx, dim), jnp.float32),
        mesh=mesh,
        scratch_types=[
            pltpu.VMEM((n_blocks, blk), jnp.int32),
            pltpu.VMEM((rows_per_w, dim), jnp.float32),
            pltpu.SemaphoreType.DMA,
            pltpu.SemaphoreType.DMA((n_blocks,)),
            pltpu.SemaphoreType.DMA((n_blocks,)),
            pltpu.SemaphoreType.DMA((n_blocks,)),
        ],
    )
    def body(tok_hbm, table_hbm, pos_hbm, out_hbm, idx_v, acc_v,
             s_idx, s_pos, s_g, s_o):
        wid = lax.axis_index("s") * _NUM_CORES + lax.axis_index("c")
        base = wid * rows_per_w
        brow = base // ctx
        col0 = lax.rem(base, ctx)

        idx_cp = [
            pltpu.async_copy(
                tok_hbm.at[brow, pl.ds(col0 + k * blk, blk)],
                idx_v.at[k], s_idx)
            for k in range(n_blocks)
        ]
        pos_cp = [
            pltpu.async_copy(
                pos_hbm.at[pl.ds(col0 + k * blk, blk)],
                acc_v.at[pl.ds(k * blk, blk)], s_pos.at[k])
            for k in range(n_blocks)
        ]
        for c in idx_cp:
            c.wait()
        g_cp = []
        for k in range(n_blocks):
            pos_cp[k].wait()
            g_cp.append(pltpu.async_copy(
                table_hbm.at[idx_v.at[k]],
                acc_v.at[pl.ds(k * blk, blk)], s_g.at[k], add=True))
        o_cp = []
        for k in range(n_blocks):
            g_cp[k].wait()
            o_cp.append(pltpu.async_copy(
                acc_v.at[pl.ds(k * blk, blk)],
                out_hbm.at[pl.ds(base + k * blk, blk)], s_o.at[k]))
        for c in o_cp:
            c.wait()

    return body


def kernel(token_ids, word_embeddings, pos_embeddings):
    batch, ctx = token_ids.shape
    _, dim = word_embeddings.shape
    rows_per_w = (batch * ctx) // _NUM_WORKERS
    n_blocks = 4
    blk = rows_per_w // n_blocks
    fn = _make_embed(batch, ctx, dim, n_blocks, blk)
    out = fn(token_ids.astype(jnp.int32), word_embeddings.astype(jnp.float32),
             pos_embeddings.astype(jnp.float32))
    return out.reshape(batch, ctx, dim)
